# baseline (speedup 1.0000x reference)
import functools

import jax
import jax.numpy as jnp
from jax import lax
from jax.experimental import pallas as pl
from jax.experimental.pallas import tpu as pltpu
from jax.experimental.pallas import tpu_sc as plsc

N = 10000
E = 320000
H = 128
G = 64
L = 3

NP_ = 10240
NCORE = 2
NSUB = 16
C = 40
PER_CORE = E // NCORE
PER_SUB = PER_CORE // NSUB
KCH = PER_SUB // C
SUB_ROWS = NP_ // NSUB
ZB = 40

_f32 = jnp.float32
_bf16 = jnp.bfloat16

def _sc_edge_body(t_hbm, cidx_hbm, didx_hbm, *rest):
    (s_out, cidx0, didx0, cidx1, didx1, rw0, rw1,
     s_sh, ci0, di0, ci1, di1, sg0, sg1) = rest

    c = lax.axis_index("c")
    s = lax.axis_index("s")

    def _zrow(r, carry):
        for q in range(H // 16):
            rw0[r, pl.ds(q * 16, 16)] = jnp.zeros((16,), _f32)
        return carry
    lax.fori_loop(0, C, _zrow, 0)

    def _zs(k, carry):
        pltpu.sync_copy(rw0.at[pl.ds(0, ZB)],
                        s_sh.at[pl.ds(s * SUB_ROWS + k * ZB, ZB)])
        return carry
    lax.fori_loop(0, SUB_ROWS // ZB, _zs, 0)

    plsc.subcore_barrier()

    w = c * NSUB + s

    def _issue_idx(k, cx, dx, sc_, sd_):
        pltpu.async_copy(cidx_hbm.at[w, k, 0], cx, sc_)
        pltpu.async_copy(didx_hbm.at[w, k, 0], dx, sd_)

    def _wait_idx(k, cx, dx, sc_, sd_):
        pltpu.make_async_copy(cidx_hbm.at[w, k, 0], cx, sc_).wait()
        pltpu.make_async_copy(didx_hbm.at[w, k, 0], dx, sd_).wait()

    def _issue_gather(cx, rw, sg):
        pltpu.async_copy(t_hbm.at[cx], rw, sg)

    def _consume(cx, dx, rw, sg):
        pltpu.make_async_copy(t_hbm.at[cx], rw, sg).wait()

        def _row(r, rc):
            for q in range(H // 16):
                sl = pl.ds(q * 16, 16)
                rw[r, sl] = jnp.maximum(rw[r, sl] + rw[C + r, sl], 0.0)
            return rc
        lax.fori_loop(0, C, _row, 0)
        pltpu.sync_copy(rw.at[pl.ds(0, C)], s_sh.at[dx], add=True)

    _issue_idx(0, cidx0, didx0, ci0, di0)
    _wait_idx(0, cidx0, didx0, ci0, di0)
    _issue_gather(cidx0, rw0, sg0)
    _issue_idx(1, cidx1, didx1, ci1, di1)

    def _pair(t, carry):
        k0 = 2 * t
        _wait_idx(k0 + 1, cidx1, didx1, ci1, di1)
        _issue_gather(cidx1, rw1, sg1)
        _consume(cidx0, didx0, rw0, sg0)

        @pl.when(k0 + 2 < KCH)
        def _():
            _issue_idx(k0 + 2, cidx0, didx0, ci0, di0)
            _wait_idx(k0 + 2, cidx0, didx0, ci0, di0)
            _issue_gather(cidx0, rw0, sg0)

        _consume(cidx1, didx1, rw1, sg1)

        @pl.when(k0 + 3 < KCH)
        def _():
            _issue_idx(k0 + 3, cidx1, didx1, ci1, di1)

        return carry
    lax.fori_loop(0, KCH // 2, _pair, 0)

    plsc.subcore_barrier()

    pltpu.sync_copy(s_sh.at[pl.ds(s * SUB_ROWS, SUB_ROWS)],
                    s_out.at[c, pl.ds(s * SUB_ROWS, SUB_ROWS)])


def _make_sc_edge():
    mesh = plsc.VectorSubcoreMesh(core_axis_name="c", subcore_axis_name="s")
    scratch = [
        pltpu.VMEM((2 * C,), jnp.int32),
        pltpu.VMEM((C,), jnp.int32),
        pltpu.VMEM((2 * C,), jnp.int32),
        pltpu.VMEM((C,), jnp.int32),
        pltpu.VMEM((2 * C, H), _f32),
        pltpu.VMEM((2 * C, H), _f32),
        pltpu.VMEM_SHARED((NP_, H), _f32),
        pltpu.SemaphoreType.DMA, pltpu.SemaphoreType.DMA,
        pltpu.SemaphoreType.DMA, pltpu.SemaphoreType.DMA,
        pltpu.SemaphoreType.DMA, pltpu.SemaphoreType.DMA,
    ]
    return pl.kernel(
        _sc_edge_body,
        out_type=(jax.ShapeDtypeStruct((NCORE, NP_, H), _f32),),
        mesh=mesh,
        scratch_types=tuple(scratch),
    )


def _sc_cnt_body(dst2d_hbm, c_out, didx, ones, zb16, c_sh):
    c = lax.axis_index("c")
    s = lax.axis_index("s")

    def _fill(r, carry):
        ones[r] = jnp.ones((16,), _f32)
        zb16[r] = jnp.zeros((16,), _f32)
        return carry
    lax.fori_loop(0, C, _fill, 0)

    def _zc(k, carry):
        pltpu.sync_copy(zb16.at[pl.ds(0, ZB)],
                        c_sh.at[pl.ds(s * SUB_ROWS + k * ZB, ZB)])
        return carry
    lax.fori_loop(0, SUB_ROWS // ZB, _zc, 0)

    plsc.subcore_barrier()

    w = c * NSUB + s
    pltpu.sync_copy(dst2d_hbm.at[w], didx)

    def _chunk(k, carry):
        pltpu.sync_copy(ones, c_sh.at[didx.at[k]], add=True)
        return carry
    lax.fori_loop(0, KCH, _chunk, 0)

    plsc.subcore_barrier()
    pltpu.sync_copy(c_sh.at[pl.ds(s * SUB_ROWS, SUB_ROWS)],
                    c_out.at[c, pl.ds(s * SUB_ROWS, SUB_ROWS)])


def _make_sc_cnt():
    mesh = plsc.VectorSubcoreMesh(core_axis_name="c", subcore_axis_name="s")
    scratch = [
        pltpu.VMEM((KCH, C), jnp.int32),
        pltpu.VMEM((C, 16), _f32),
        pltpu.VMEM((C, 16), _f32),
        pltpu.VMEM_SHARED((NP_, 16), _f32),
    ]
    return pl.kernel(
        _sc_cnt_body,
        out_type=(jax.ShapeDtypeStruct((NCORE, NP_, 16), _f32),),
        mesh=mesh,
        scratch_types=tuple(scratch),
    )


BLK = 512
_DOT = functools.partial(jnp.dot, preferred_element_type=_f32)


def _tc_pre_body(x_ref, posp_ref, nw0, nb0, nw1, nb1, w0a, w0b, w0p, b0,
                 a_out, b_out):
    h = _DOT(jnp.maximum(_DOT(x_ref[...], nw0[...]) + nb0[...], 0.0),
             nw1[...]) + nb1[...]
    a_out[...] = _DOT(h, w0a[...])
    b_out[...] = _DOT(h, w0b[...]) + _DOT(posp_ref[...], w0p[...]) + b0[...]


def _tc_mid_body(s0, s1, c0, c1, w1, b1, gw0, gb0, gw1, gb1,
                 w0a, w0b, w0p, b0, posp_ref, a_out, b_out):
    agg = _DOT(s0[...] + s1[...], w1[...]) + (c0[...] + c1[...]) * b1[...]
    t = jnp.maximum(_DOT(agg, gw0[...]) + gb0[...], 0.0)
    h = jnp.maximum(_DOT(t, gw1[...]) + gb1[...], 0.0)
    a_out[...] = _DOT(h, w0a[...])
    b_out[...] = _DOT(h, w0b[...]) + _DOT(posp_ref[...], w0p[...]) + b0[...]


def _tc_last_body(s0, s1, c0, c1, w1, b1, gw0, gb0, gw1, gb1,
                  l1w, l1b, l2w, l2b, batch_ref, out_ref):
    agg = _DOT(s0[...] + s1[...], w1[...]) + (c0[...] + c1[...]) * b1[...]
    t = jnp.maximum(_DOT(agg, gw0[...]) + gb0[...], 0.0)
    h = jnp.maximum(_DOT(t, gw1[...]) + gb1[...], 0.0)
    f = _DOT(jnp.maximum(_DOT(h, l1w[...]) + l1b[...], 0.0), l2w[...]) + l2b[...]
    gids = lax.broadcasted_iota(jnp.int32, (BLK, G), 1)
    onehot = (batch_ref[...] == gids).astype(_f32)
    contrib = lax.dot_general(onehot, f, (((0,), (0,)), ((), ())),
                              preferred_element_type=_f32)

    @pl.when(pl.program_id(0) == 0)
    def _():
        out_ref[...] = jnp.zeros_like(out_ref)

    out_ref[...] += contrib


def _row_spec(width):
    return pl.BlockSpec((BLK, width), lambda i: (i, 0))


def _full_spec(shape):
    nd = len(shape)
    return pl.BlockSpec(shape, lambda i: (0,) * nd)


def _tc_pre(xp, posp, nw0, nb0, nw1, nb1, w0a, w0b, w0p, b0):
    grid = (NP_ // BLK,)
    return pl.pallas_call(
        _tc_pre_body,
        grid=grid,
        in_specs=[_row_spec(H), _row_spec(H)] + [
            _full_spec(a.shape) for a in (nw0, nb0, nw1, nb1, w0a, w0b, w0p, b0)],
        out_specs=[_row_spec(H), _row_spec(H)],
        out_shape=[jax.ShapeDtypeStruct((NP_, H), _f32)] * 2,
    )(xp, posp, nw0, nb0, nw1, nb1, w0a, w0b, w0p, b0)


def _tc_mid(s0, s1, c0, c1, w1, b1, gw0, gb0, gw1, gb1,
            w0a, w0b, w0p, b0, posp):
    grid = (NP_ // BLK,)
    return pl.pallas_call(
        _tc_mid_body,
        grid=grid,
        in_specs=[_row_spec(H), _row_spec(H), _row_spec(1), _row_spec(1)] + [
            _full_spec(a.shape)
            for a in (w1, b1, gw0, gb0, gw1, gb1, w0a, w0b, w0p, b0)] + [
            _row_spec(H)],
        out_specs=[_row_spec(H), _row_spec(H)],
        out_shape=[jax.ShapeDtypeStruct((NP_, H), _f32)] * 2,
    )(s0, s1, c0, c1, w1, b1, gw0, gb0, gw1, gb1, w0a, w0b, w0p, b0, posp)


def _tc_last(s0, s1, c0, c1, w1, b1, gw0, gb0, gw1, gb1,
             l1w, l1b, l2w, l2b, batch_p):
    grid = (NP_ // BLK,)
    return pl.pallas_call(
        _tc_last_body,
        grid=grid,
        in_specs=[_row_spec(H), _row_spec(H), _row_spec(1), _row_spec(1)] + [
            _full_spec(a.shape)
            for a in (w1, b1, gw0, gb0, gw1, gb1, l1w, l1b, l2w, l2b)] + [
            _row_spec(1)],
        out_specs=pl.BlockSpec((G, H), lambda i: (0, 0)),
        out_shape=jax.ShapeDtypeStruct((G, H), _f32),
    )(s0, s1, c0, c1, w1, b1, gw0, gb0, gw1, gb1, l1w, l1b, l2w, l2b, batch_p)


def kernel(x, pos, edge_index, batch,
           node_w0, node_b0, node_w1, node_b1,
           loc_w0, loc_b0, loc_w1, loc_b1,
           glob_w0, glob_b0, glob_w1, glob_b1,
           lin1_w, lin1_b, lin2_w, lin2_b):
    pad = NP_ - N
    src = edge_index[0]
    dst = edge_index[1]
    dst3d = dst.reshape(NCORE * NSUB, KCH, C)
    cidx4d = jnp.concatenate(
        [dst.reshape(-1, C), src.reshape(-1, C) + NP_],
        axis=1).reshape(NCORE * NSUB, KCH, 1, 2 * C)
    didx4d = dst.reshape(NCORE * NSUB, KCH, 1, C)
    xp = jnp.pad(x, ((0, pad), (0, 0)))
    posp = jnp.pad(pos, ((0, pad), (0, H - 3)))
    batch_p = jnp.pad(batch, (0, pad), constant_values=G).reshape(NP_, 1)

    w0a = loc_w0[:, :H, :]
    w0b = loc_w0[:, H:2 * H, :]
    w0p = jnp.pad(loc_w0[:, 2 * H:, :], ((0, 0), (0, H - 3), (0, 0)))
    w1p = loc_w1
    b0 = loc_b0.reshape(L, 1, H)
    b1 = loc_b1.reshape(L, 1, H)
    gb0 = glob_b0.reshape(L, 1, H)
    gb1 = glob_b1.reshape(L, 1, H)
    nb0 = node_b0.reshape(1, H)
    nb1 = node_b1.reshape(1, H)
    l1b = lin1_b.reshape(1, H // 2)
    l2b = lin2_b.reshape(1, H)

    sc_edge = _make_sc_edge()
    sc_cnt = _make_sc_cnt()

    (c_parts,) = sc_cnt(dst3d)
    cnt0 = c_parts[0, :, 0:1]
    cnt1 = c_parts[1, :, 0:1]

    a0, b0t = _tc_pre(xp, posp, node_w0, nb0, node_w1, nb1,
                      w0a[0], w0b[0], w0p[0], b0[0])
    (s_parts,) = sc_edge(jnp.concatenate([a0, b0t], 0), cidx4d, didx4d)

    a1, b1t = _tc_mid(s_parts[0], s_parts[1], cnt0, cnt1,
                      w1p[0], b1[0], glob_w0[0], gb0[0], glob_w1[0], gb1[0],
                      w0a[1], w0b[1], w0p[1], b0[1], posp)
    (s_parts1,) = sc_edge(jnp.concatenate([a1, b1t], 0), cidx4d, didx4d)

    a2, b2t = _tc_mid(s_parts1[0], s_parts1[1], cnt0, cnt1,
                      w1p[1], b1[1], glob_w0[1], gb0[1], glob_w1[1], gb1[1],
                      w0a[2], w0b[2], w0p[2], b0[2], posp)
    (s_parts2,) = sc_edge(jnp.concatenate([a2, b2t], 0), cidx4d, didx4d)

    out = _tc_last(s_parts2[0], s_parts2[1], cnt0, cnt1,
                   w1p[2], b1[2], glob_w0[2], gb0[2], glob_w1[2], gb1[2],
                   lin1_w, l1b, lin2_w, l2b, batch_p)
    return out


# speedup vs baseline: 1.0831x; 1.0831x over previous
import functools

import jax
import jax.numpy as jnp
from jax import lax
from jax.experimental import pallas as pl
from jax.experimental.pallas import tpu as pltpu
from jax.experimental.pallas import tpu_sc as plsc

N = 10000
E = 320000
H = 128
G = 64
L = 3

NP_ = 10240
NCORE = 2
NSUB = 16
C = 40
PER_CORE = E // NCORE
PER_SUB = PER_CORE // NSUB
KCH = PER_SUB // C
SUB_ROWS = NP_ // NSUB
ZB = 40

_f32 = jnp.float32
_bf16 = jnp.bfloat16

def _sc_edge_body(a_hbm, b_hbm, src_hbm, dst_hbm, *rest):
    (s_out, sidx0, didx0, sidx1, didx1, arow0, brow0, arow1, brow1,
     s_sh, si0, di0, si1, di1, sa0, sb0, sa1, sb1) = rest

    c = lax.axis_index("c")
    s = lax.axis_index("s")

    def _zrow(r, carry):
        for q in range(H // 16):
            arow0[r, pl.ds(q * 16, 16)] = jnp.zeros((16,), _f32)
        return carry
    lax.fori_loop(0, C, _zrow, 0)

    def _zs(k, carry):
        pltpu.sync_copy(arow0.at[pl.ds(0, ZB)],
                        s_sh.at[pl.ds(s * SUB_ROWS + k * ZB, ZB)])
        return carry
    lax.fori_loop(0, SUB_ROWS // ZB, _zs, 0)

    plsc.subcore_barrier()

    base0 = c * PER_CORE + s * PER_SUB

    def _src_slice(k):
        return src_hbm.at[pl.ds(base0 + k * C, C)]

    def _dst_slice(k):
        return dst_hbm.at[pl.ds(base0 + k * C, C)]

    def _row_loop(ar, br):
        def _row(r, rc):
            for q in range(H // 16):
                sl = pl.ds(q * 16, 16)
                ar[r, sl] = jnp.maximum(ar[r, sl] + br[r, sl], 0.0)
            return rc
        lax.fori_loop(0, C, _row, 0)

    def _step(k, sx, dx, ar, br, ss, ds_, sa, sb, have_next):
        pltpu.make_async_copy(a_hbm.at[dx], ar, sa).wait()
        pltpu.make_async_copy(b_hbm.at[sx], br, sb).wait()
        if have_next:
            pltpu.async_copy(_src_slice(k + 2), sx, ss)
        _row_loop(ar, br)
        if have_next:
            pltpu.make_async_copy(_src_slice(k + 2), sx, ss).wait()
            pltpu.async_copy(b_hbm.at[sx], br, sb)
        pltpu.sync_copy(ar, s_sh.at[dx], add=True)
        if have_next:
            pltpu.async_copy(_dst_slice(k + 2), dx, ds_)
            pltpu.make_async_copy(_dst_slice(k + 2), dx, ds_).wait()
            pltpu.async_copy(a_hbm.at[dx], ar, sa)

    def _prime(k, sx, dx, ss, ds_, ar, br, sa, sb):
        pltpu.async_copy(_src_slice(k), sx, ss)
        pltpu.async_copy(_dst_slice(k), dx, ds_)
        pltpu.make_async_copy(_src_slice(k), sx, ss).wait()
        pltpu.make_async_copy(_dst_slice(k), dx, ds_).wait()
        pltpu.async_copy(a_hbm.at[dx], ar, sa)
        pltpu.async_copy(b_hbm.at[sx], br, sb)

    _prime(0, sidx0, didx0, si0, di0, arow0, brow0, sa0, sb0)
    _prime(1, sidx1, didx1, si1, di1, arow1, brow1, sa1, sb1)

    def _pair(t, carry):
        k0 = 2 * t

        @pl.when(k0 + 2 < KCH)
        def _():
            _step(k0, sidx0, didx0, arow0, brow0, si0, di0, sa0, sb0, True)
            _step(k0 + 1, sidx1, didx1, arow1, brow1, si1, di1, sa1, sb1,
                  True)

        @pl.when(k0 + 2 >= KCH)
        def _():
            _step(k0, sidx0, didx0, arow0, brow0, si0, di0, sa0, sb0, False)
            _step(k0 + 1, sidx1, didx1, arow1, brow1, si1, di1, sa1, sb1,
                  False)

        return carry
    lax.fori_loop(0, KCH // 2, _pair, 0)

    plsc.subcore_barrier()

    pltpu.sync_copy(s_sh.at[pl.ds(s * SUB_ROWS, SUB_ROWS)],
                    s_out.at[c, pl.ds(s * SUB_ROWS, SUB_ROWS)])


def _make_sc_edge():
    mesh = plsc.VectorSubcoreMesh(core_axis_name="c", subcore_axis_name="s")
    scratch = [
        pltpu.VMEM((C,), jnp.int32),
        pltpu.VMEM((C,), jnp.int32),
        pltpu.VMEM((C,), jnp.int32),
        pltpu.VMEM((C,), jnp.int32),
        pltpu.VMEM((C, H), _f32),
        pltpu.VMEM((C, H), _f32),
        pltpu.VMEM((C, H), _f32),
        pltpu.VMEM((C, H), _f32),
        pltpu.VMEM_SHARED((NP_, H), _f32),
        pltpu.SemaphoreType.DMA, pltpu.SemaphoreType.DMA,
        pltpu.SemaphoreType.DMA, pltpu.SemaphoreType.DMA,
        pltpu.SemaphoreType.DMA, pltpu.SemaphoreType.DMA,
        pltpu.SemaphoreType.DMA, pltpu.SemaphoreType.DMA,
    ]
    return pl.kernel(
        _sc_edge_body,
        out_type=(jax.ShapeDtypeStruct((NCORE, NP_, H), _f32),),
        mesh=mesh,
        scratch_types=tuple(scratch),
    )


def _sc_cnt_body(dst2d_hbm, c_out, didx, ones, zb16, c_sh):
    c = lax.axis_index("c")
    s = lax.axis_index("s")

    def _fill(r, carry):
        ones[r] = jnp.ones((16,), _f32)
        zb16[r] = jnp.zeros((16,), _f32)
        return carry
    lax.fori_loop(0, C, _fill, 0)

    def _zc(k, carry):
        pltpu.sync_copy(zb16.at[pl.ds(0, ZB)],
                        c_sh.at[pl.ds(s * SUB_ROWS + k * ZB, ZB)])
        return carry
    lax.fori_loop(0, SUB_ROWS // ZB, _zc, 0)

    plsc.subcore_barrier()

    w = c * NSUB + s
    pltpu.sync_copy(dst2d_hbm.at[w], didx)

    def _chunk(k, carry):
        pltpu.sync_copy(ones, c_sh.at[didx.at[k]], add=True)
        return carry
    lax.fori_loop(0, KCH, _chunk, 0)

    plsc.subcore_barrier()
    pltpu.sync_copy(c_sh.at[pl.ds(s * SUB_ROWS, SUB_ROWS)],
                    c_out.at[c, pl.ds(s * SUB_ROWS, SUB_ROWS)])


def _make_sc_cnt():
    mesh = plsc.VectorSubcoreMesh(core_axis_name="c", subcore_axis_name="s")
    scratch = [
        pltpu.VMEM((KCH, C), jnp.int32),
        pltpu.VMEM((C, 16), _f32),
        pltpu.VMEM((C, 16), _f32),
        pltpu.VMEM_SHARED((NP_, 16), _f32),
    ]
    return pl.kernel(
        _sc_cnt_body,
        out_type=(jax.ShapeDtypeStruct((NCORE, NP_, 16), _f32),),
        mesh=mesh,
        scratch_types=tuple(scratch),
    )


BLK = 512
_DOT = functools.partial(jnp.dot, preferred_element_type=_f32)


def _tc_pre_body(x_ref, posp_ref, nw0, nb0, nw1, nb1, w0a, w0b, w0p, b0,
                 a_out, b_out):
    h = _DOT(jnp.maximum(_DOT(x_ref[...], nw0[...]) + nb0[...], 0.0),
             nw1[...]) + nb1[...]
    a_out[...] = _DOT(h, w0a[...])
    b_out[...] = _DOT(h, w0b[...]) + _DOT(posp_ref[...], w0p[...]) + b0[...]


def _tc_mid_body(s0, s1, c0, c1, w1, b1, gw0, gb0, gw1, gb1,
                 w0a, w0b, w0p, b0, posp_ref, a_out, b_out):
    agg = _DOT(s0[...] + s1[...], w1[...]) + (c0[...] + c1[...]) * b1[...]
    t = jnp.maximum(_DOT(agg, gw0[...]) + gb0[...], 0.0)
    h = jnp.maximum(_DOT(t, gw1[...]) + gb1[...], 0.0)
    a_out[...] = _DOT(h, w0a[...])
    b_out[...] = _DOT(h, w0b[...]) + _DOT(posp_ref[...], w0p[...]) + b0[...]


def _tc_last_body(s0, s1, c0, c1, w1, b1, gw0, gb0, gw1, gb1,
                  l1w, l1b, l2w, l2b, batch_ref, out_ref):
    agg = _DOT(s0[...] + s1[...], w1[...]) + (c0[...] + c1[...]) * b1[...]
    t = jnp.maximum(_DOT(agg, gw0[...]) + gb0[...], 0.0)
    h = jnp.maximum(_DOT(t, gw1[...]) + gb1[...], 0.0)
    f = _DOT(jnp.maximum(_DOT(h, l1w[...]) + l1b[...], 0.0), l2w[...]) + l2b[...]
    gids = lax.broadcasted_iota(jnp.int32, (BLK, G), 1)
    onehot = (batch_ref[...] == gids).astype(_f32)
    contrib = lax.dot_general(onehot, f, (((0,), (0,)), ((), ())),
                              preferred_element_type=_f32)

    @pl.when(pl.program_id(0) == 0)
    def _():
        out_ref[...] = jnp.zeros_like(out_ref)

    out_ref[...] += contrib


def _row_spec(width):
    return pl.BlockSpec((BLK, width), lambda i: (i, 0))


def _full_spec(shape):
    nd = len(shape)
    return pl.BlockSpec(shape, lambda i: (0,) * nd)


def _tc_pre(xp, posp, nw0, nb0, nw1, nb1, w0a, w0b, w0p, b0):
    grid = (NP_ // BLK,)
    return pl.pallas_call(
        _tc_pre_body,
        grid=grid,
        in_specs=[_row_spec(H), _row_spec(H)] + [
            _full_spec(a.shape) for a in (nw0, nb0, nw1, nb1, w0a, w0b, w0p, b0)],
        out_specs=[_row_spec(H), _row_spec(H)],
        out_shape=[jax.ShapeDtypeStruct((NP_, H), _f32)] * 2,
    )(xp, posp, nw0, nb0, nw1, nb1, w0a, w0b, w0p, b0)


def _tc_mid(s0, s1, c0, c1, w1, b1, gw0, gb0, gw1, gb1,
            w0a, w0b, w0p, b0, posp):
    grid = (NP_ // BLK,)
    return pl.pallas_call(
        _tc_mid_body,
        grid=grid,
        in_specs=[_row_spec(H), _row_spec(H), _row_spec(1), _row_spec(1)] + [
            _full_spec(a.shape)
            for a in (w1, b1, gw0, gb0, gw1, gb1, w0a, w0b, w0p, b0)] + [
            _row_spec(H)],
        out_specs=[_row_spec(H), _row_spec(H)],
        out_shape=[jax.ShapeDtypeStruct((NP_, H), _f32)] * 2,
    )(s0, s1, c0, c1, w1, b1, gw0, gb0, gw1, gb1, w0a, w0b, w0p, b0, posp)


def _tc_last(s0, s1, c0, c1, w1, b1, gw0, gb0, gw1, gb1,
             l1w, l1b, l2w, l2b, batch_p):
    grid = (NP_ // BLK,)
    return pl.pallas_call(
        _tc_last_body,
        grid=grid,
        in_specs=[_row_spec(H), _row_spec(H), _row_spec(1), _row_spec(1)] + [
            _full_spec(a.shape)
            for a in (w1, b1, gw0, gb0, gw1, gb1, l1w, l1b, l2w, l2b)] + [
            _row_spec(1)],
        out_specs=pl.BlockSpec((G, H), lambda i: (0, 0)),
        out_shape=jax.ShapeDtypeStruct((G, H), _f32),
    )(s0, s1, c0, c1, w1, b1, gw0, gb0, gw1, gb1, l1w, l1b, l2w, l2b, batch_p)


def kernel(x, pos, edge_index, batch,
           node_w0, node_b0, node_w1, node_b1,
           loc_w0, loc_b0, loc_w1, loc_b1,
           glob_w0, glob_b0, glob_w1, glob_b1,
           lin1_w, lin1_b, lin2_w, lin2_b):
    pad = NP_ - N
    src = edge_index[0]
    dst = edge_index[1]
    dst3d = dst.reshape(NCORE * NSUB, KCH, C)
    xp = jnp.pad(x, ((0, pad), (0, 0)))
    posp = jnp.pad(pos, ((0, pad), (0, H - 3)))
    batch_p = jnp.pad(batch, (0, pad), constant_values=G).reshape(NP_, 1)

    w0a = loc_w0[:, :H, :]
    w0b = loc_w0[:, H:2 * H, :]
    w0p = jnp.pad(loc_w0[:, 2 * H:, :], ((0, 0), (0, H - 3), (0, 0)))
    w1p = loc_w1
    b0 = loc_b0.reshape(L, 1, H)
    b1 = loc_b1.reshape(L, 1, H)
    gb0 = glob_b0.reshape(L, 1, H)
    gb1 = glob_b1.reshape(L, 1, H)
    nb0 = node_b0.reshape(1, H)
    nb1 = node_b1.reshape(1, H)
    l1b = lin1_b.reshape(1, H // 2)
    l2b = lin2_b.reshape(1, H)

    sc_edge = _make_sc_edge()
    sc_cnt = _make_sc_cnt()

    (c_parts,) = sc_cnt(dst3d)
    cnt0 = c_parts[0, :, 0:1]
    cnt1 = c_parts[1, :, 0:1]

    a0, b0t = _tc_pre(xp, posp, node_w0, nb0, node_w1, nb1,
                      w0a[0], w0b[0], w0p[0], b0[0])
    (s_parts,) = sc_edge(a0, b0t, src, dst)

    a1, b1t = _tc_mid(s_parts[0], s_parts[1], cnt0, cnt1,
                      w1p[0], b1[0], glob_w0[0], gb0[0], glob_w1[0], gb1[0],
                      w0a[1], w0b[1], w0p[1], b0[1], posp)
    (s_parts1,) = sc_edge(a1, b1t, src, dst)

    a2, b2t = _tc_mid(s_parts1[0], s_parts1[1], cnt0, cnt1,
                      w1p[1], b1[1], glob_w0[1], gb0[1], glob_w1[1], gb1[1],
                      w0a[2], w0b[2], w0p[2], b0[2], posp)
    (s_parts2,) = sc_edge(a2, b2t, src, dst)

    out = _tc_last(s_parts2[0], s_parts2[1], cnt0, cnt1,
                   w1p[2], b1[2], glob_w0[2], gb0[2], glob_w1[2], gb1[2],
                   lin1_w, l1b, lin2_w, l2b, batch_p)
    return out


# speedup vs baseline: 1.1103x; 1.0251x over previous
import functools

import jax
import jax.numpy as jnp
from jax import lax
from jax.experimental import pallas as pl
from jax.experimental.pallas import tpu as pltpu
from jax.experimental.pallas import tpu_sc as plsc

N = 10000
E = 320000
H = 128
G = 64
L = 3

NP_ = 10240
NCORE = 2
NSUB = 16
C = 40
PER_CORE = E // NCORE
PER_SUB = PER_CORE // NSUB
KCH = PER_SUB // C
SUB_ROWS = NP_ // NSUB
ZB = 40

_f32 = jnp.float32
_bf16 = jnp.bfloat16

def _sc_edge_body(a_hbm, b_hbm, src_hbm, dst_hbm, *rest):
    (s_out, sidx0, didx0, sidx1, didx1, arow0, brow0, arow1, brow1,
     s_sh, si0, di0, si1, di1, sa0, sb0, sa1, sb1) = rest

    c = lax.axis_index("c")
    s = lax.axis_index("s")

    def _zrow(r, carry):
        for q in range(H // 16):
            arow0[r, pl.ds(q * 16, 16)] = jnp.zeros((16,), _f32)
        return carry
    lax.fori_loop(0, C, _zrow, 0)

    def _zs(k, carry):
        pltpu.sync_copy(arow0.at[pl.ds(0, ZB)],
                        s_sh.at[pl.ds(s * SUB_ROWS + k * ZB, ZB)])
        return carry
    lax.fori_loop(0, SUB_ROWS // ZB, _zs, 0)

    plsc.subcore_barrier()

    base0 = c * PER_CORE + s * PER_SUB

    def _issue_idx(k, sx, dx, ss, ds_):
        pltpu.async_copy(src_hbm.at[pl.ds(base0 + k * C, C)], sx, ss)
        pltpu.async_copy(dst_hbm.at[pl.ds(base0 + k * C, C)], dx, ds_)

    def _wait_idx(k, sx, dx, ss, ds_):
        pltpu.make_async_copy(src_hbm.at[pl.ds(base0 + k * C, C)], sx, ss).wait()
        pltpu.make_async_copy(dst_hbm.at[pl.ds(base0 + k * C, C)], dx, ds_).wait()

    def _issue_gather(sx, dx, ar, br, sa, sb):
        pltpu.async_copy(a_hbm.at[dx], ar, sa)
        pltpu.async_copy(b_hbm.at[sx], br, sb)

    def _consume(sx, dx, ar, br, sa, sb):
        pltpu.make_async_copy(a_hbm.at[dx], ar, sa).wait()
        pltpu.make_async_copy(b_hbm.at[sx], br, sb).wait()

        @plsc.parallel_loop(0, C)
        def _row(r):
            for q in range(H // 16):
                sl = pl.ds(q * 16, 16)
                ar[r, sl] = jnp.maximum(ar[r, sl] + br[r, sl], 0.0)

        pltpu.sync_copy(ar, s_sh.at[dx], add=True)

    _issue_idx(0, sidx0, didx0, si0, di0)
    _wait_idx(0, sidx0, didx0, si0, di0)
    _issue_gather(sidx0, didx0, arow0, brow0, sa0, sb0)
    _issue_idx(1, sidx1, didx1, si1, di1)

    def _pair(t, carry):
        k0 = 2 * t
        _wait_idx(k0 + 1, sidx1, didx1, si1, di1)
        _issue_gather(sidx1, didx1, arow1, brow1, sa1, sb1)
        _consume(sidx0, didx0, arow0, brow0, sa0, sb0)

        @pl.when(k0 + 2 < KCH)
        def _():
            _issue_idx(k0 + 2, sidx0, didx0, si0, di0)
            _wait_idx(k0 + 2, sidx0, didx0, si0, di0)
            _issue_gather(sidx0, didx0, arow0, brow0, sa0, sb0)

        _consume(sidx1, didx1, arow1, brow1, sa1, sb1)

        @pl.when(k0 + 3 < KCH)
        def _():
            _issue_idx(k0 + 3, sidx1, didx1, si1, di1)

        return carry
    lax.fori_loop(0, KCH // 2, _pair, 0)

    plsc.subcore_barrier()

    pltpu.sync_copy(s_sh.at[pl.ds(s * SUB_ROWS, SUB_ROWS)],
                    s_out.at[c, pl.ds(s * SUB_ROWS, SUB_ROWS)])


def _make_sc_edge():
    mesh = plsc.VectorSubcoreMesh(core_axis_name="c", subcore_axis_name="s")
    scratch = [
        pltpu.VMEM((C,), jnp.int32),
        pltpu.VMEM((C,), jnp.int32),
        pltpu.VMEM((C,), jnp.int32),
        pltpu.VMEM((C,), jnp.int32),
        pltpu.VMEM((C, H), _f32),
        pltpu.VMEM((C, H), _f32),
        pltpu.VMEM((C, H), _f32),
        pltpu.VMEM((C, H), _f32),
        pltpu.VMEM_SHARED((NP_, H), _f32),
        pltpu.SemaphoreType.DMA, pltpu.SemaphoreType.DMA,
        pltpu.SemaphoreType.DMA, pltpu.SemaphoreType.DMA,
        pltpu.SemaphoreType.DMA, pltpu.SemaphoreType.DMA,
        pltpu.SemaphoreType.DMA, pltpu.SemaphoreType.DMA,
    ]
    return pl.kernel(
        _sc_edge_body,
        out_type=(jax.ShapeDtypeStruct((NCORE, NP_, H), _f32),),
        mesh=mesh,
        scratch_types=tuple(scratch),
    )


def _sc_cnt_body(dst2d_hbm, c_out, didx, ones, zb16, c_sh):
    c = lax.axis_index("c")
    s = lax.axis_index("s")

    def _fill(r, carry):
        ones[r] = jnp.ones((16,), _f32)
        zb16[r] = jnp.zeros((16,), _f32)
        return carry
    lax.fori_loop(0, C, _fill, 0)

    def _zc(k, carry):
        pltpu.sync_copy(zb16.at[pl.ds(0, ZB)],
                        c_sh.at[pl.ds(s * SUB_ROWS + k * ZB, ZB)])
        return carry
    lax.fori_loop(0, SUB_ROWS // ZB, _zc, 0)

    plsc.subcore_barrier()

    w = c * NSUB + s
    pltpu.sync_copy(dst2d_hbm.at[w], didx)

    def _chunk(k, carry):
        pltpu.sync_copy(ones, c_sh.at[didx.at[k]], add=True)
        return carry
    lax.fori_loop(0, KCH, _chunk, 0)

    plsc.subcore_barrier()
    pltpu.sync_copy(c_sh.at[pl.ds(s * SUB_ROWS, SUB_ROWS)],
                    c_out.at[c, pl.ds(s * SUB_ROWS, SUB_ROWS)])


def _make_sc_cnt():
    mesh = plsc.VectorSubcoreMesh(core_axis_name="c", subcore_axis_name="s")
    scratch = [
        pltpu.VMEM((KCH, C), jnp.int32),
        pltpu.VMEM((C, 16), _f32),
        pltpu.VMEM((C, 16), _f32),
        pltpu.VMEM_SHARED((NP_, 16), _f32),
    ]
    return pl.kernel(
        _sc_cnt_body,
        out_type=(jax.ShapeDtypeStruct((NCORE, NP_, 16), _f32),),
        mesh=mesh,
        scratch_types=tuple(scratch),
    )


BLK = 512
_DOT = functools.partial(jnp.dot, preferred_element_type=_f32)


def _tc_pre_body(x_ref, posp_ref, nw0, nb0, nw1, nb1, w0a, w0b, w0p, b0,
                 a_out, b_out):
    h = _DOT(jnp.maximum(_DOT(x_ref[...], nw0[...]) + nb0[...], 0.0),
             nw1[...]) + nb1[...]
    a_out[...] = _DOT(h, w0a[...])
    b_out[...] = _DOT(h, w0b[...]) + _DOT(posp_ref[...], w0p[...]) + b0[...]


def _tc_mid_body(s0, s1, c0, c1, w1, b1, gw0, gb0, gw1, gb1,
                 w0a, w0b, w0p, b0, posp_ref, a_out, b_out):
    agg = _DOT(s0[...] + s1[...], w1[...]) + (c0[...] + c1[...]) * b1[...]
    t = jnp.maximum(_DOT(agg, gw0[...]) + gb0[...], 0.0)
    h = jnp.maximum(_DOT(t, gw1[...]) + gb1[...], 0.0)
    a_out[...] = _DOT(h, w0a[...])
    b_out[...] = _DOT(h, w0b[...]) + _DOT(posp_ref[...], w0p[...]) + b0[...]


def _tc_last_body(s0, s1, c0, c1, w1, b1, gw0, gb0, gw1, gb1,
                  l1w, l1b, l2w, l2b, batch_ref, out_ref):
    agg = _DOT(s0[...] + s1[...], w1[...]) + (c0[...] + c1[...]) * b1[...]
    t = jnp.maximum(_DOT(agg, gw0[...]) + gb0[...], 0.0)
    h = jnp.maximum(_DOT(t, gw1[...]) + gb1[...], 0.0)
    f = _DOT(jnp.maximum(_DOT(h, l1w[...]) + l1b[...], 0.0), l2w[...]) + l2b[...]
    gids = lax.broadcasted_iota(jnp.int32, (BLK, G), 1)
    onehot = (batch_ref[...] == gids).astype(_f32)
    contrib = lax.dot_general(onehot, f, (((0,), (0,)), ((), ())),
                              preferred_element_type=_f32)

    @pl.when(pl.program_id(0) == 0)
    def _():
        out_ref[...] = jnp.zeros_like(out_ref)

    out_ref[...] += contrib


def _row_spec(width):
    return pl.BlockSpec((BLK, width), lambda i: (i, 0))


def _full_spec(shape):
    nd = len(shape)
    return pl.BlockSpec(shape, lambda i: (0,) * nd)


def _tc_pre(xp, posp, nw0, nb0, nw1, nb1, w0a, w0b, w0p, b0):
    grid = (NP_ // BLK,)
    return pl.pallas_call(
        _tc_pre_body,
        grid=grid,
        in_specs=[_row_spec(H), _row_spec(H)] + [
            _full_spec(a.shape) for a in (nw0, nb0, nw1, nb1, w0a, w0b, w0p, b0)],
        out_specs=[_row_spec(H), _row_spec(H)],
        out_shape=[jax.ShapeDtypeStruct((NP_, H), _f32)] * 2,
    )(xp, posp, nw0, nb0, nw1, nb1, w0a, w0b, w0p, b0)


def _tc_mid(s0, s1, c0, c1, w1, b1, gw0, gb0, gw1, gb1,
            w0a, w0b, w0p, b0, posp):
    grid = (NP_ // BLK,)
    return pl.pallas_call(
        _tc_mid_body,
        grid=grid,
        in_specs=[_row_spec(H), _row_spec(H), _row_spec(1), _row_spec(1)] + [
            _full_spec(a.shape)
            for a in (w1, b1, gw0, gb0, gw1, gb1, w0a, w0b, w0p, b0)] + [
            _row_spec(H)],
        out_specs=[_row_spec(H), _row_spec(H)],
        out_shape=[jax.ShapeDtypeStruct((NP_, H), _f32)] * 2,
    )(s0, s1, c0, c1, w1, b1, gw0, gb0, gw1, gb1, w0a, w0b, w0p, b0, posp)


def _tc_last(s0, s1, c0, c1, w1, b1, gw0, gb0, gw1, gb1,
             l1w, l1b, l2w, l2b, batch_p):
    grid = (NP_ // BLK,)
    return pl.pallas_call(
        _tc_last_body,
        grid=grid,
        in_specs=[_row_spec(H), _row_spec(H), _row_spec(1), _row_spec(1)] + [
            _full_spec(a.shape)
            for a in (w1, b1, gw0, gb0, gw1, gb1, l1w, l1b, l2w, l2b)] + [
            _row_spec(1)],
        out_specs=pl.BlockSpec((G, H), lambda i: (0, 0)),
        out_shape=jax.ShapeDtypeStruct((G, H), _f32),
    )(s0, s1, c0, c1, w1, b1, gw0, gb0, gw1, gb1, l1w, l1b, l2w, l2b, batch_p)


def kernel(x, pos, edge_index, batch,
           node_w0, node_b0, node_w1, node_b1,
           loc_w0, loc_b0, loc_w1, loc_b1,
           glob_w0, glob_b0, glob_w1, glob_b1,
           lin1_w, lin1_b, lin2_w, lin2_b):
    pad = NP_ - N
    src = edge_index[0]
    dst = edge_index[1]
    dst3d = dst.reshape(NCORE * NSUB, KCH, C)
    xp = jnp.pad(x, ((0, pad), (0, 0)))
    posp = jnp.pad(pos, ((0, pad), (0, H - 3)))
    batch_p = jnp.pad(batch, (0, pad), constant_values=G).reshape(NP_, 1)

    w0a = loc_w0[:, :H, :]
    w0b = loc_w0[:, H:2 * H, :]
    w0p = jnp.pad(loc_w0[:, 2 * H:, :], ((0, 0), (0, H - 3), (0, 0)))
    w1p = loc_w1
    b0 = loc_b0.reshape(L, 1, H)
    b1 = loc_b1.reshape(L, 1, H)
    gb0 = glob_b0.reshape(L, 1, H)
    gb1 = glob_b1.reshape(L, 1, H)
    nb0 = node_b0.reshape(1, H)
    nb1 = node_b1.reshape(1, H)
    l1b = lin1_b.reshape(1, H // 2)
    l2b = lin2_b.reshape(1, H)

    sc_edge = _make_sc_edge()
    sc_cnt = _make_sc_cnt()

    (c_parts,) = sc_cnt(dst3d)
    cnt0 = c_parts[0, :, 0:1]
    cnt1 = c_parts[1, :, 0:1]

    a0, b0t = _tc_pre(xp, posp, node_w0, nb0, node_w1, nb1,
                      w0a[0], w0b[0], w0p[0], b0[0])
    (s_parts,) = sc_edge(a0, b0t, src, dst)

    a1, b1t = _tc_mid(s_parts[0], s_parts[1], cnt0, cnt1,
                      w1p[0], b1[0], glob_w0[0], gb0[0], glob_w1[0], gb1[0],
                      w0a[1], w0b[1], w0p[1], b0[1], posp)
    (s_parts1,) = sc_edge(a1, b1t, src, dst)

    a2, b2t = _tc_mid(s_parts1[0], s_parts1[1], cnt0, cnt1,
                      w1p[1], b1[1], glob_w0[1], gb0[1], glob_w1[1], gb1[1],
                      w0a[2], w0b[2], w0p[2], b0[2], posp)
    (s_parts2,) = sc_edge(a2, b2t, src, dst)

    out = _tc_last(s_parts2[0], s_parts2[1], cnt0, cnt1,
                   w1p[2], b1[2], glob_w0[2], gb0[2], glob_w1[2], gb1[2],
                   lin1_w, l1b, lin2_w, l2b, batch_p)
    return out
